# Initial kernel scaffold; baseline (speedup 1.0000x reference)
import jax
import jax.numpy as jnp
from jax.experimental import pallas as pl


def kernel(source, motions):
    raise NotImplementedError("write your pallas kernel here")


# speedup vs baseline: 1.2403x; 1.2403x over previous
import functools

import jax
import jax.numpy as jnp
from jax import lax
from jax.experimental import pallas as pl
from jax.experimental.pallas import tpu as pltpu
from jax.experimental.pallas import tpu_sc as plsc

H = 64
W = 64
C = 128
NKP1 = 11
BS = 8
N = BS * NKP1 * H * W
NW = 32
PER_W = N // NW
CH = 64
NCHUNK = PER_W // CH
LANES = 16


def _prep_body(mx_ref, my_ref, inw_ref, ine_ref, isw_ref, ise_ref,
               wnw_ref, wne_ref, wsw_ref, wse_ref):
    gx = mx_ref[...]
    gy = my_ref[...]
    x = (gx + 1.0) * (W / 2.0) - 0.5
    y = (gy + 1.0) * (H / 2.0) - 0.5
    xw = jnp.floor(x)
    yn = jnp.floor(y)
    fx = x - xw
    fy = y - yn
    xwi = xw.astype(jnp.int32)
    yni = yn.astype(jnp.int32)
    xei = xwi + 1
    ysi = yni + 1
    w_m = (xwi > -1) & (xwi < W)
    e_m = (xei > -1) & (xei < W)
    n_m = (yni > -1) & (yni < H)
    s_m = (ysi > -1) & (ysi < H)
    e = 1.0 - fx
    s = 1.0 - fy
    wnw_ref[...] = s * e * (w_m & n_m).astype(jnp.float32)
    wne_ref[...] = s * fx * (e_m & n_m).astype(jnp.float32)
    wsw_ref[...] = fy * e * (w_m & s_m).astype(jnp.float32)
    wse_ref[...] = fy * fx * (e_m & s_m).astype(jnp.float32)
    xwc = jnp.clip(xwi, 0, W - 1)
    xec = jnp.clip(xei, 0, W - 1)
    ync = jnp.clip(yni, 0, H - 1)
    ysc = jnp.clip(ysi, 0, H - 1)
    inw_ref[...] = ync * W + xwc
    ine_ref[...] = ync * W + xec
    isw_ref[...] = ysc * W + xwc
    ise_ref[...] = ysc * W + xec


def _prep(mx, my):
    shp = mx.shape
    outs = [jax.ShapeDtypeStruct(shp, jnp.int32)] * 4 + \
           [jax.ShapeDtypeStruct(shp, jnp.float32)] * 4
    return pl.pallas_call(_prep_body, out_shape=outs)(mx, my)


def _sc_body(tab, inw, ine, isw, ise, wnw, wne, wsw, wse, out,
             idx_v, w_v, rows_v, out_v, sem):
    cid = lax.axis_index("c")
    sid = lax.axis_index("s")
    wid = sid * 2 + cid

    def chunk(g, carry):
        base = (wid * NCHUNK + g) * CH
        for k, ref in enumerate((inw, ine, isw, ise)):
            pltpu.sync_copy(ref.at[pl.ds(base, CH)], idx_v.at[k])
        for k, ref in enumerate((wnw, wne, wsw, wse)):
            pltpu.sync_copy(ref.at[pl.ds(base, CH)], w_v.at[k])
        cps = [pltpu.async_copy(tab.at[idx_v.at[k]], rows_v.at[k], sem)
               for k in range(4)]
        for cp in cps:
            cp.wait()
        for i0 in range(0, CH, LANES):
            wv0 = w_v[0, pl.ds(i0, LANES)]
            wv1 = w_v[1, pl.ds(i0, LANES)]
            wv2 = w_v[2, pl.ds(i0, LANES)]
            wv3 = w_v[3, pl.ds(i0, LANES)]
            for ii in range(LANES):
                i = i0 + ii
                w0 = wv0[ii]
                w1 = wv1[ii]
                w2 = wv2[ii]
                w3 = wv3[ii]
                for j in range(C // LANES):
                    sl = pl.ds(j * LANES, LANES)
                    acc = w0 * rows_v[0, i, sl]
                    acc = acc + w1 * rows_v[1, i, sl]
                    acc = acc + w2 * rows_v[2, i, sl]
                    acc = acc + w3 * rows_v[3, i, sl]
                    out_v[i, sl] = acc
        pltpu.sync_copy(out_v, out.at[pl.ds(base, CH)])
        return carry

    lax.fori_loop(0, NCHUNK, chunk, 0)


@functools.partial(
    pl.kernel,
    out_type=jax.ShapeDtypeStruct((N, C), jnp.float32),
    mesh=plsc.VectorSubcoreMesh(core_axis_name="c", subcore_axis_name="s"),
    scratch_types=[
        pltpu.VMEM((4, CH), jnp.int32),
        pltpu.VMEM((4, CH), jnp.float32),
        pltpu.VMEM((4, CH, C), jnp.float32),
        pltpu.VMEM((CH, C), jnp.float32),
        pltpu.SemaphoreType.DMA,
    ],
)
def _sc_sample(tab, inw, ine, isw, ise, wnw, wne, wsw, wse, out,
               idx_v, w_v, rows_v, out_v, sem):
    _sc_body(tab, inw, ine, isw, ise, wnw, wne, wsw, wse, out,
             idx_v, w_v, rows_v, out_v, sem)


def kernel(source, motions):
    bs = motions.shape[0]
    mx = motions[..., 0].reshape(-1, C)
    my = motions[..., 1].reshape(-1, C)
    inw, ine, isw, ise, wnw, wne, wsw, wse = _prep(mx, my)
    flat = lambda a: a.reshape(-1)
    table = source.reshape(H * W, C)
    out = _sc_sample(table,
                     flat(inw), flat(ine), flat(isw), flat(ise),
                     flat(wnw), flat(wne), flat(wsw), flat(wse))
    return out.reshape(bs, NKP1, H, W, C)


# speedup vs baseline: 1.5227x; 1.2276x over previous
import functools

import jax
import jax.numpy as jnp
from jax import lax
from jax.experimental import pallas as pl
from jax.experimental.pallas import tpu as pltpu
from jax.experimental.pallas import tpu_sc as plsc

H = 64
W = 64
C = 128
NKP1 = 11
BS = 8
N = BS * NKP1 * H * W
NW = 32
PER_W = N // NW
CH = 64
NCHUNK = PER_W // CH
LANES = 16


def _prep_body(mx_ref, my_ref, inw_ref, ine_ref, isw_ref, ise_ref,
               wnw_ref, wne_ref, wsw_ref, wse_ref):
    gx = mx_ref[...]
    gy = my_ref[...]
    x = (gx + 1.0) * (W / 2.0) - 0.5
    y = (gy + 1.0) * (H / 2.0) - 0.5
    xw = jnp.floor(x)
    yn = jnp.floor(y)
    fx = x - xw
    fy = y - yn
    xwi = xw.astype(jnp.int32)
    yni = yn.astype(jnp.int32)
    xei = xwi + 1
    ysi = yni + 1
    w_m = (xwi > -1) & (xwi < W)
    e_m = (xei > -1) & (xei < W)
    n_m = (yni > -1) & (yni < H)
    s_m = (ysi > -1) & (ysi < H)
    e = 1.0 - fx
    s = 1.0 - fy
    wnw_ref[...] = s * e * (w_m & n_m).astype(jnp.float32)
    wne_ref[...] = s * fx * (e_m & n_m).astype(jnp.float32)
    wsw_ref[...] = fy * e * (w_m & s_m).astype(jnp.float32)
    wse_ref[...] = fy * fx * (e_m & s_m).astype(jnp.float32)
    xwc = jnp.clip(xwi, 0, W - 1)
    xec = jnp.clip(xei, 0, W - 1)
    ync = jnp.clip(yni, 0, H - 1)
    ysc = jnp.clip(ysi, 0, H - 1)
    inw_ref[...] = ync * W + xwc
    ine_ref[...] = ync * W + xec
    isw_ref[...] = ysc * W + xwc
    ise_ref[...] = ysc * W + xec


def _prep(mx, my):
    shp = mx.shape
    outs = [jax.ShapeDtypeStruct(shp, jnp.int32)] * 4 + \
           [jax.ShapeDtypeStruct(shp, jnp.float32)] * 4
    return pl.pallas_call(_prep_body, out_shape=outs)(mx, my)


def _sc_body(tab, inw, ine, isw, ise, wnw, wne, wsw, wse, out,
             idx_v, w_v, rows_v, outb_v, idx_sem, rows_sem, out_sem):
    cid = lax.axis_index("c")
    sid = lax.axis_index("s")
    wid = sid * 2 + cid
    idx_refs = (inw, ine, isw, ise)
    w_refs = (wnw, wne, wsw, wse)

    def chunk_base(c):
        return (wid * NCHUNK + c) * CH

    def prefetch_idx(c, slot):
        base = chunk_base(c)
        for k in range(4):
            pltpu.async_copy(idx_refs[k].at[pl.ds(base, CH)],
                             idx_v.at[slot, k], idx_sem)

    def prefetch_w(c, slot):
        base = chunk_base(c)
        for k in range(4):
            pltpu.async_copy(w_refs[k].at[pl.ds(base, CH)],
                             w_v.at[slot, k], idx_sem)

    prefetch_idx(0, 0)
    prefetch_w(0, 0)

    def step(g, carry):
        cur = lax.rem(g, 2)
        prv = 1 - cur

        @pl.when(g >= 1)
        def _():
            for k in range(4):
                pltpu.make_async_copy(tab.at[pl.ds(0, CH)],
                                      rows_v.at[prv, k], rows_sem).wait()

        @pl.when(g < NCHUNK)
        def _():
            for k in range(4):
                pltpu.make_async_copy(inw.at[pl.ds(0, CH)],
                                      idx_v.at[cur, k], idx_sem).wait()
                pltpu.make_async_copy(wnw.at[pl.ds(0, CH)],
                                      w_v.at[cur, k], idx_sem).wait()
            for k in range(4):
                pltpu.async_copy(tab.at[idx_v.at[cur, k]],
                                 rows_v.at[cur, k], rows_sem)

        @pl.when(g + 1 < NCHUNK)
        def _():
            prefetch_idx(g + 1, prv)

        @pl.when(g >= 2)
        def _():
            pltpu.make_async_copy(tab.at[pl.ds(0, CH)],
                                  outb_v.at[cur], out_sem).wait()

        @pl.when(g >= 1)
        def _():
            for i0 in range(0, CH, LANES):
                wv0 = w_v[prv, 0, pl.ds(i0, LANES)]
                wv1 = w_v[prv, 1, pl.ds(i0, LANES)]
                wv2 = w_v[prv, 2, pl.ds(i0, LANES)]
                wv3 = w_v[prv, 3, pl.ds(i0, LANES)]
                for ii in range(LANES):
                    i = i0 + ii
                    w0 = wv0[ii]
                    w1 = wv1[ii]
                    w2 = wv2[ii]
                    w3 = wv3[ii]
                    for j in range(C // LANES):
                        sl = pl.ds(j * LANES, LANES)
                        acc = w0 * rows_v[prv, 0, i, sl]
                        acc = acc + w1 * rows_v[prv, 1, i, sl]
                        acc = acc + w2 * rows_v[prv, 2, i, sl]
                        acc = acc + w3 * rows_v[prv, 3, i, sl]
                        outb_v[prv, i, sl] = acc
            pltpu.async_copy(outb_v.at[prv],
                             out.at[pl.ds(chunk_base(g - 1), CH)], out_sem)

        @pl.when(g + 1 < NCHUNK)
        def _():
            prefetch_w(g + 1, prv)

        return carry

    lax.fori_loop(0, NCHUNK + 1, step, 0)
    pltpu.make_async_copy(tab.at[pl.ds(0, CH)], outb_v.at[0], out_sem).wait()


@functools.partial(
    pl.kernel,
    out_type=jax.ShapeDtypeStruct((N, C), jnp.float32),
    mesh=plsc.VectorSubcoreMesh(core_axis_name="c", subcore_axis_name="s"),
    scratch_types=[
        pltpu.VMEM((2, 4, CH), jnp.int32),
        pltpu.VMEM((2, 4, CH), jnp.float32),
        pltpu.VMEM((2, 4, CH, C), jnp.float32),
        pltpu.VMEM((2, CH, C), jnp.float32),
        pltpu.SemaphoreType.DMA,
        pltpu.SemaphoreType.DMA,
        pltpu.SemaphoreType.DMA,
    ],
)
def _sc_sample(tab, inw, ine, isw, ise, wnw, wne, wsw, wse, out,
               idx_v, w_v, rows_v, outb_v, idx_sem, rows_sem, out_sem):
    _sc_body(tab, inw, ine, isw, ise, wnw, wne, wsw, wse, out,
             idx_v, w_v, rows_v, outb_v, idx_sem, rows_sem, out_sem)


def kernel(source, motions):
    bs = motions.shape[0]
    mx = motions[..., 0].reshape(-1, C)
    my = motions[..., 1].reshape(-1, C)
    inw, ine, isw, ise, wnw, wne, wsw, wse = _prep(mx, my)
    flat = lambda a: a.reshape(-1)
    table = source.reshape(H * W, C)
    out = _sc_sample(table,
                     flat(inw), flat(ine), flat(isw), flat(ise),
                     flat(wnw), flat(wne), flat(wsw), flat(wse))
    return out.reshape(bs, NKP1, H, W, C)
